# Initial kernel scaffold; baseline (speedup 1.0000x reference)
import jax
import jax.numpy as jnp
from jax.experimental import pallas as pl


def kernel(x, pos_edges, neg_edges, W1, a1_src, a1_dst, W2, a2_src, a2_dst, nodes):
    raise NotImplementedError("write your pallas kernel here")


# speedup vs baseline: 8.4750x; 8.4750x over previous
import functools

import jax
import jax.numpy as jnp
from jax import lax
from jax.experimental import pallas as pl
from jax.experimental.pallas import tpu as pltpu
from jax.experimental.pallas import tpu_sc as plsc

N = 10000
D_H = 64
HEADS = 4
ALPHA = 0.2

NP = 10240
NR = 8192
ROWS_PER_SUB = NR // 16
CHUNK = 128
NC = 2
NSUB = 16
DEN_PER_SUB = NP // 16
NW = NC * NSUB
DP = 2 * D_H


def _pad_edges(edges, e_pad):
    e = edges.shape[1]
    src = jnp.concatenate([edges[0].astype(jnp.int32),
                           jnp.zeros((e_pad - e,), jnp.int32)])
    dst = jnp.concatenate([edges[1].astype(jnp.int32),
                           jnp.zeros((e_pad - e,), jnp.int32)])
    valid = jnp.concatenate([jnp.ones((e,), jnp.float32),
                             jnp.zeros((e_pad - e,), jnp.float32)])
    return src, dst, valid


def _proj_body(x_ref, w_ref, asr_ref, adt_ref, hw_ref, es_ref, ed_ref):
    xb = x_ref[...]
    for s in range(2):
        for h in range(HEADS):
            m = jnp.dot(xb, w_ref[s, h], preferred_element_type=jnp.float32)
            hw_ref[s, h // 2, :, (h % 2) * D_H:(h % 2) * D_H + D_H] = m
            es_ref[s, h] = jnp.sum(m * asr_ref[s, h][None, :], axis=1)
            ed_ref[s, h] = jnp.sum(m * adt_ref[s, h][None, :], axis=1)


def _project(x_pad, W, a_src, a_dst):
    din = x_pad.shape[1]
    tile = 1024
    return pl.pallas_call(
        _proj_body,
        grid=(NP // tile,),
        in_specs=[
            pl.BlockSpec((tile, din), lambda i: (i, 0)),
            pl.BlockSpec((2, HEADS, din, D_H), lambda i: (0, 0, 0, 0)),
            pl.BlockSpec((2, HEADS, D_H), lambda i: (0, 0, 0)),
            pl.BlockSpec((2, HEADS, D_H), lambda i: (0, 0, 0)),
        ],
        out_specs=[
            pl.BlockSpec((2, 2, tile, DP), lambda i: (0, 0, i, 0)),
            pl.BlockSpec((2, HEADS, tile), lambda i: (0, 0, i)),
            pl.BlockSpec((2, HEADS, tile), lambda i: (0, 0, i)),
        ],
        out_shape=[
            jax.ShapeDtypeStruct((2, 2, NP, DP), jnp.float32),
            jax.ShapeDtypeStruct((2, HEADS, NP), jnp.float32),
            jax.ShapeDtypeStruct((2, HEADS, NP), jnp.float32),
        ],
    )(x_pad, W, a_src, a_dst)


def _sc_edge(src, dst, valid, hw_flat, es, ed, zero_rows, zero_den, n_sub):
    per_tec = n_sub * CHUNK
    mesh = plsc.VectorSubcoreMesh(core_axis_name="c", subcore_axis_name="s")

    @functools.partial(
        pl.kernel,
        mesh=mesh,
        compiler_params=pltpu.CompilerParams(needs_layout_passes=False),
        out_type=[
            jax.ShapeDtypeStruct((NC, 2, 2, NR, DP), jnp.float32),
            jax.ShapeDtypeStruct((NC, HEADS, NP), jnp.float32),
        ],
        scratch_types=[
            pltpu.VMEM((CHUNK,), jnp.int32),
            pltpu.VMEM((CHUNK,), jnp.int32),
            pltpu.VMEM((CHUNK,), jnp.int32),
            pltpu.VMEM((CHUNK,), jnp.float32),
            pltpu.VMEM((CHUNK,), jnp.float32),
            pltpu.VMEM((CHUNK,), jnp.float32),
            pltpu.VMEM((CHUNK,), jnp.float32),
            pltpu.VMEM((CHUNK,), jnp.float32),
            pltpu.VMEM((CHUNK,), jnp.int32),
            pltpu.VMEM((CHUNK, DP), jnp.float32),
            pltpu.VMEM((NP,), jnp.float32),
            pltpu.VMEM((NP,), jnp.float32),
            pltpu.VMEM((NP,), jnp.float32),
            pltpu.VMEM((NP,), jnp.float32),
            pltpu.VMEM_SHARED((NR, DP), jnp.float32),
            pltpu.VMEM_SHARED((NP,), jnp.float32),
            pltpu.VMEM_SHARED((NP,), jnp.float32),
            pltpu.SemaphoreType.DMA,
        ],
    )
    def k(src_h, dst_h, val_h, hw_h, es_h, ed_h, zr_h, zd_h,
          num_o, den_o,
          src_v, dst_v, dstc_v, val_v, w0_v, w1_v, wf0_v, wf1_v,
          idx_v, rows_v, es0_v, ed0_v, es1_v, ed1_v,
          num_s, den0_s, den1_s, sem):
        cid = lax.axis_index("c")
        sid = lax.axis_index("s")
        wid = sid * NC + cid
        my_rows = sid * ROWS_PER_SUB
        my_den = sid * DEN_PER_SUB

        for p in range(2):
            pltpu.sync_copy(es_h.at[2 * p], es0_v)
            pltpu.sync_copy(ed_h.at[2 * p], ed0_v)
            pltpu.sync_copy(es_h.at[2 * p + 1], es1_v)
            pltpu.sync_copy(ed_h.at[2 * p + 1], ed1_v)
            for r in range(2):
                if r == 0:
                    pltpu.sync_copy(
                        zd_h, den0_s.at[pl.ds(my_den, DEN_PER_SUB)])
                    pltpu.sync_copy(
                        zd_h, den1_s.at[pl.ds(my_den, DEN_PER_SUB)])
                for j in range(ROWS_PER_SUB // CHUNK):
                    pltpu.sync_copy(
                        zr_h, num_s.at[pl.ds(my_rows + j * CHUNK, CHUNK)])
                plsc.subcore_barrier()

                def sub_body(t, carry):
                    base = wid * per_tec + t * CHUNK
                    pltpu.sync_copy(src_h.at[pl.ds(base, CHUNK)], src_v)
                    pltpu.sync_copy(dst_h.at[pl.ds(base, CHUNK)], dst_v)
                    pltpu.sync_copy(val_h.at[pl.ds(base, CHUNK)], val_v)
                    for g in range(CHUNK // 16):
                        sl = pl.ds(g * 16, 16)
                        isrc = src_v[sl]
                        idst = dst_v[sl]
                        vv = val_v[sl]
                        a0 = plsc.load_gather(es0_v, [isrc])
                        b0 = plsc.load_gather(ed0_v, [idst])
                        z0 = a0 + b0
                        l0 = jnp.where(z0 >= 0.0, z0, ALPHA * z0)
                        wf0 = jnp.exp(l0) * vv
                        a1 = plsc.load_gather(es1_v, [isrc])
                        b1 = plsc.load_gather(ed1_v, [idst])
                        z1 = a1 + b1
                        l1 = jnp.where(z1 >= 0.0, z1, ALPHA * z1)
                        wf1 = jnp.exp(l1) * vv
                        dw = idst - r * NR
                        inwin = (dw >= 0) & (dw < NR)
                        fwin = jnp.where(inwin, 1.0, 0.0)
                        dstc_v[sl] = jnp.clip(dw, 0, NR - 1)
                        w0_v[sl] = wf0 * fwin
                        w1_v[sl] = wf1 * fwin
                        if r == 0:
                            wf0_v[sl] = wf0
                            wf1_v[sl] = wf1
                        idx_v[sl] = isrc + p * NP
                    pltpu.async_copy(hw_h.at[idx_v], rows_v, sem).wait()

                    def scale_body(e, c2):
                        esp = jnp.broadcast_to(e, (16,))
                        wsp0 = plsc.load_gather(w0_v, [esp])
                        wsp1 = plsc.load_gather(w1_v, [esp])
                        for kk in range(DP // 16):
                            col = lax.iota(jnp.int32, 16) + kk * 16
                            wsp = wsp0 if kk < DP // 32 else wsp1
                            rr = plsc.load_gather(rows_v, [esp, col])
                            plsc.store_scatter(rows_v, [esp, col], rr * wsp)
                        return c2

                    lax.fori_loop(0, CHUNK, scale_body, 0)
                    if r == 0:
                        pltpu.sync_copy(wf0_v, den0_s.at[dst_v], add=True)
                        pltpu.sync_copy(wf1_v, den1_s.at[dst_v], add=True)
                    pltpu.sync_copy(rows_v, num_s.at[dstc_v], add=True)
                    return carry

                lax.fori_loop(0, n_sub, sub_body, 0)
                plsc.subcore_barrier()
                pltpu.sync_copy(
                    num_s.at[pl.ds(my_rows, ROWS_PER_SUB)],
                    num_o.at[cid, p, r, pl.ds(my_rows, ROWS_PER_SUB)])
                if r == 0:
                    pltpu.sync_copy(
                        den0_s.at[pl.ds(my_den, DEN_PER_SUB)],
                        den_o.at[cid, 2 * p, pl.ds(my_den, DEN_PER_SUB)])
                    pltpu.sync_copy(
                        den1_s.at[pl.ds(my_den, DEN_PER_SUB)],
                        den_o.at[cid, 2 * p + 1, pl.ds(my_den, DEN_PER_SUB)])
                plsc.subcore_barrier()

    return k(src, dst, valid, hw_flat, es, ed, zero_rows, zero_den)


def _epi_body(concat, np_ref, dp_ref, nn_ref, dn_ref, out_ref):
    def agg(n_ref, d_ref):
        outs = []
        for h in range(HEADS):
            c0 = (h % 2) * D_H
            num = (n_ref[0, h // 2, 0, :, c0:c0 + D_H]
                   + n_ref[1, h // 2, 0, :, c0:c0 + D_H])
            den = d_ref[0, h] + d_ref[1, h]
            outs.append(num / (den[:, None] + 1e-16))
        if concat:
            return jnp.concatenate(outs, axis=1)
        return (outs[0] + outs[1] + outs[2] + outs[3]) / HEADS

    d = agg(np_ref, dp_ref) - agg(nn_ref, dn_ref)
    out_ref[...] = jnp.where(d > 0.0, d, jnp.exp(d) - 1.0)


def _epilogue(num_p, den_p, num_n, den_n, concat):
    tile = 1024
    dout = HEADS * D_H if concat else D_H

    def nmap(i):
        return (0, 0, i // (NR // tile), i % (NR // tile), 0)

    return pl.pallas_call(
        functools.partial(_epi_body, concat),
        grid=(NP // tile,),
        in_specs=[
            pl.BlockSpec((NC, 2, 1, tile, DP), nmap),
            pl.BlockSpec((NC, HEADS, tile), lambda i: (0, 0, i)),
            pl.BlockSpec((NC, 2, 1, tile, DP), nmap),
            pl.BlockSpec((NC, HEADS, tile), lambda i: (0, 0, i)),
        ],
        out_specs=pl.BlockSpec((tile, dout), lambda i: (i, 0)),
        out_shape=jax.ShapeDtypeStruct((NP, dout), jnp.float32),
    )(num_p, den_p, num_n, den_n)


def _layer(x_pad, edge_data, W, a_src, a_dst, concat):
    hw, es, ed = _project(x_pad, W, a_src, a_dst)
    zero_rows = jnp.zeros((CHUNK, DP), jnp.float32)
    zero_den = jnp.zeros((DEN_PER_SUB,), jnp.float32)
    parts = []
    for s in range(2):
        src, dst, valid, n_sub = edge_data[s]
        hw_flat = hw[s].reshape(2 * NP, DP)
        num, den = _sc_edge(src, dst, valid, hw_flat, es[s], ed[s],
                            zero_rows, zero_den, n_sub)
        parts.append((num, den))
    (num_p, den_p), (num_n, den_n) = parts
    return _epilogue(num_p, den_p, num_n, den_n, concat)


def kernel(x, pos_edges, neg_edges, W1, a1_src, a1_dst, W2, a2_src, a2_dst,
           nodes):
    blk = NW * CHUNK
    e_pos_pad = ((pos_edges.shape[1] + blk - 1) // blk) * blk
    e_neg_pad = ((neg_edges.shape[1] + blk - 1) // blk) * blk
    sp, dp, vp = _pad_edges(pos_edges, e_pos_pad)
    sn, dn, vn = _pad_edges(neg_edges, e_neg_pad)
    edge_data = [
        (sp, dp, vp, e_pos_pad // blk),
        (sn, dn, vn, e_neg_pad // blk),
    ]

    x_pad = jnp.zeros((NP, x.shape[1]), jnp.float32).at[:N].set(x)
    h1 = _layer(x_pad, edge_data, W1, a1_src, a1_dst, True)
    h2 = _layer(h1, edge_data, W2, a2_src, a2_dst, False)
    return h2[nodes]
